# Initial kernel scaffold; baseline (speedup 1.0000x reference)
import jax
import jax.numpy as jnp
from jax.experimental import pallas as pl


def kernel(diag, tril, row, col):
    raise NotImplementedError("write your pallas kernel here")


# speedup vs baseline: 3.9370x; 3.9370x over previous
import functools

import jax
import jax.numpy as jnp
from jax import lax
from jax.experimental import pallas as pl
from jax.experimental.pallas import tpu as pltpu
from jax.experimental.pallas import tpu_sc as plsc

_NUM_CORES = 2
_NUM_SUBCORES = 16
_NW = _NUM_CORES * _NUM_SUBCORES
_LANES = 16


def _tc_diag_body(diag_ref, dsi_ref, dm_ref):
    x = diag_ref[...]
    xp1 = x + 1.0
    dsi_ref[...] = lax.rsqrt(xp1)
    dm_ref[...] = x / xp1


def _compute_dsi(diag):
    n, d = diag.shape
    flat = diag.reshape(-1)
    total = flat.shape[0]
    assert total % 128 == 0
    rows = total // 128
    diag2d = flat.reshape(rows, 128)
    out_sds = jax.ShapeDtypeStruct((rows, 128), jnp.float32)
    dsi2d, dm2d = pl.pallas_call(
        _tc_diag_body,
        out_shape=(out_sds, out_sds),
    )(diag2d)
    return dsi2d.reshape(n, d), dm2d.reshape(n, d)


def _sc_edge_kernel(n_nodes, e_edges, d, chunk):
    ew = e_edges // _NW
    nchunks = ew // chunk
    fchunk = chunk * d
    nvec = fchunk // _LANES

    mesh = plsc.VectorSubcoreMesh(
        core_axis_name="c", subcore_axis_name="s",
        num_cores=_NUM_CORES, num_subcores=_NUM_SUBCORES,
    )

    @functools.partial(
        pl.kernel,
        mesh=mesh,
        out_type=jax.ShapeDtypeStruct((e_edges * d,), jnp.float32),
        compiler_params=pltpu.CompilerParams(
            use_tc_tiling_on_sc=False, needs_layout_passes=False),
        scratch_types=[
            pltpu.VMEM((chunk,), jnp.int32),
            pltpu.VMEM((chunk,), jnp.int32),
            pltpu.VMEM((chunk, 8), jnp.float32),
            pltpu.VMEM((chunk, 8), jnp.float32),
            pltpu.VMEM((fchunk,), jnp.float32),
            pltpu.VMEM((fchunk,), jnp.float32),
            pltpu.SemaphoreType.DMA,
            pltpu.SemaphoreType.DMA,
        ],
    )
    def edge_kernel(dsi_hbm, trilf_hbm, row_hbm, col_hbm, outf_hbm,
                    row_v, col_v, left_v, right_v, tril_v, out_v,
                    sem_l, sem_r):
        wid = lax.axis_index("s") * _NUM_CORES + lax.axis_index("c")
        epv = _LANES // d
        iota = lax.iota(jnp.int32, _LANES)
        q = lax.shift_right_logical(iota, 2)
        r = lax.bitwise_and(iota, 3)

        def chunk_body(g, carry):
            base = pl.multiple_of(wid * ew + g * chunk, 8)
            fbase = pl.multiple_of(base * d, 8)
            pltpu.sync_copy(row_hbm.at[pl.ds(base, chunk)], row_v)
            pltpu.sync_copy(col_hbm.at[pl.ds(base, chunk)], col_v)
            cpl = pltpu.async_copy(dsi_hbm.at[row_v], left_v, sem_l)
            cpr = pltpu.async_copy(dsi_hbm.at[col_v], right_v, sem_r)
            pltpu.sync_copy(trilf_hbm.at[pl.ds(fbase, fchunk)], tril_v)
            cpl.wait()
            cpr.wait()

            def inner(i, c2):
                rows = q + lax.broadcast(i * epv, (_LANES,))
                lv = plsc.load_gather(left_v, [rows, r])
                rv = plsc.load_gather(right_v, [rows, r])
                t = tril_v[pl.ds(i * _LANES, _LANES)]
                out_v[pl.ds(i * _LANES, _LANES)] = lv * t * rv
                return c2

            lax.fori_loop(0, nvec, inner, 0)
            pltpu.sync_copy(out_v, outf_hbm.at[pl.ds(fbase, fchunk)])
            return carry

        lax.fori_loop(0, nchunks, chunk_body, 0)

    return edge_kernel


def kernel(diag, tril, row, col):
    n, d = diag.shape
    e = tril.shape[0]
    dsi, diag_maps = _compute_dsi(diag)
    dsi8 = jnp.zeros((n, 8), jnp.float32).at[:, :d].set(dsi)

    chunk = 2000
    assert e % (_NW * chunk) == 0

    edge_kernel = _sc_edge_kernel(n, e, d, chunk)
    outf = edge_kernel(dsi8, tril.reshape(-1), row, col)
    return diag_maps, outf.reshape(e, d)


# speedup vs baseline: 53.4792x; 13.5838x over previous
import functools

import jax
import jax.numpy as jnp
from jax import lax
from jax.experimental import pallas as pl
from jax.experimental.pallas import tpu as pltpu
from jax.experimental.pallas import tpu_sc as plsc

_NUM_CORES = 2
_NUM_SUBCORES = 16
_NW = _NUM_CORES * _NUM_SUBCORES
_LANES = 16
_GRP = 128
_UNROLL = 8


def _tc_diag_body(diag_ref, dsi_ref, dm_ref):
    x = diag_ref[...]
    xp1 = x + 1.0
    dsi_ref[...] = lax.rsqrt(xp1)
    dm_ref[...] = x / xp1


def _compute_dsi(diag):
    n, d = diag.shape
    flat = diag.reshape(-1)
    total = flat.shape[0]
    assert total % 128 == 0
    rows = total // 128
    diag2d = flat.reshape(rows, 128)
    out_sds = jax.ShapeDtypeStruct((rows, 128), jnp.float32)
    dsi2d, dm2d = pl.pallas_call(
        _tc_diag_body,
        out_shape=(out_sds, out_sds),
    )(diag2d)
    return dsi2d.reshape(n, d), dm2d.reshape(n, d)


def _sc_edge_kernel(e_edges, d, cgrp):
    g_total = e_edges // _GRP
    chunk = cgrp * _GRP
    fchunk = chunk * d
    nvec = fchunk // _LANES
    nout = nvec // _UNROLL
    nchunks = -(-g_total // (_NW * cgrp))

    mesh = plsc.VectorSubcoreMesh(
        core_axis_name="c", subcore_axis_name="s",
        num_cores=_NUM_CORES, num_subcores=_NUM_SUBCORES,
    )

    @functools.partial(
        pl.kernel,
        mesh=mesh,
        out_type=jax.ShapeDtypeStruct((e_edges * d,), jnp.float32),
        compiler_params=pltpu.CompilerParams(
            use_tc_tiling_on_sc=False, needs_layout_passes=False),
        scratch_types=[
            pltpu.VMEM((2, chunk), jnp.int32),
            pltpu.VMEM((2, chunk), jnp.int32),
            pltpu.VMEM((2, chunk, 8), jnp.float32),
            pltpu.VMEM((2, chunk, 8), jnp.float32),
            pltpu.VMEM((2, fchunk), jnp.float32),
            pltpu.VMEM((2, fchunk), jnp.float32),
            pltpu.SemaphoreType.DMA,
            pltpu.SemaphoreType.DMA,
            pltpu.SemaphoreType.DMA,
            pltpu.SemaphoreType.DMA,
            pltpu.SemaphoreType.DMA,
            pltpu.SemaphoreType.DMA,
            pltpu.SemaphoreType.DMA,
            pltpu.SemaphoreType.DMA,
        ],
    )
    def edge_kernel(dsi_hbm, trilf_hbm, row_hbm, col_hbm, outf_hbm,
                    row_v, col_v, left_v, right_v, tril_v, out_v,
                    sem_i0, sem_i1, sem_g0, sem_g1,
                    sem_t0, sem_t1, sem_o0, sem_o1):
        wid = lax.axis_index("s") * _NUM_CORES + lax.axis_index("c")
        iota = lax.iota(jnp.int32, _LANES)
        sem_i = (sem_i0, sem_i1)
        sem_g = (sem_g0, sem_g1)
        sem_t = (sem_t0, sem_t1)
        sem_o = (sem_o0, sem_o1)

        def chunk_base(c):
            gs = jnp.minimum((wid * nchunks + c) * cgrp, g_total - cgrp)
            return pl.multiple_of(gs * _GRP, 8)

        def issue_idx(c, b):
            base = chunk_base(c)
            pltpu.async_copy(row_hbm.at[pl.ds(base, chunk)],
                             row_v.at[b], sem_i[b])
            pltpu.async_copy(col_hbm.at[pl.ds(base, chunk)],
                             col_v.at[b], sem_i[b])

        def wait_idx(b):
            pltpu.make_async_copy(row_hbm.at[pl.ds(0, chunk)],
                                  row_v.at[b], sem_i[b]).wait()
            pltpu.make_async_copy(col_hbm.at[pl.ds(0, chunk)],
                                  col_v.at[b], sem_i[b]).wait()

        def issue_data(c, b):
            fbase = pl.multiple_of(chunk_base(c) * d, 8)
            pltpu.async_copy(dsi_hbm.at[row_v.at[b]], left_v.at[b], sem_g[b])
            pltpu.async_copy(dsi_hbm.at[col_v.at[b]], right_v.at[b], sem_g[b])
            pltpu.async_copy(trilf_hbm.at[pl.ds(fbase, fchunk)],
                             tril_v.at[b], sem_t[b])

        def wait_data(b):
            pltpu.make_async_copy(dsi_hbm.at[row_v.at[b]],
                                  left_v.at[b], sem_g[b]).wait()
            pltpu.make_async_copy(dsi_hbm.at[col_v.at[b]],
                                  right_v.at[b], sem_g[b]).wait()
            pltpu.make_async_copy(trilf_hbm.at[pl.ds(0, fchunk)],
                                  tril_v.at[b], sem_t[b]).wait()

        def issue_out(c, b):
            fbase = pl.multiple_of(chunk_base(c) * d, 8)
            pltpu.async_copy(out_v.at[b],
                             outf_hbm.at[pl.ds(fbase, fchunk)], sem_o[b])

        def wait_out(b):
            pltpu.make_async_copy(out_v.at[b],
                                  outf_hbm.at[pl.ds(0, fchunk)], sem_o[b]).wait()

        def compute(b):
            lvb = left_v.at[b]
            rvb = right_v.at[b]
            tvb = tril_v.at[b]
            ovb = out_v.at[b]

            def step(k, c2):
                comp = lax.bitwise_and(k, 3)
                elbase = lax.shift_left(lax.shift_right_logical(k, 2), 7)
                cols = lax.broadcast(comp, (_LANES,))
                fb = k * (_UNROLL * _LANES)
                for u in range(_UNROLL):
                    rows = lax.broadcast(elbase + u * _LANES, (_LANES,)) + iota
                    lv = plsc.load_gather(lvb, [rows, cols])
                    rv = plsc.load_gather(rvb, [rows, cols])
                    t = tvb[pl.ds(fb + u * _LANES, _LANES)]
                    ovb[pl.ds(fb + u * _LANES, _LANES)] = lv * t * rv
                return c2

            lax.fori_loop(0, nout, step, 0)

        issue_idx(0, 0)
        wait_idx(0)
        issue_data(0, 0)
        issue_idx(1, 1)

        def pipe_step(c, b):
            bp = 1 - b
            wait_idx(b)
            issue_data(c, b)
            wait_data(bp)

            @pl.when(c + 1 < nchunks)
            def _():
                issue_idx(c + 1, bp)

            @pl.when(c >= 3)
            def _():
                wait_out(bp)

            compute(bp)
            issue_out(c - 1, bp)

        def body(j, carry):
            c1 = 2 * j + 1
            pipe_step(c1, 1)

            @pl.when(c1 + 1 < nchunks)
            def _():
                pipe_step(c1 + 1, 0)

            return carry

        lax.fori_loop(0, -(-(nchunks - 1) // 2), body, 0)

        bl = (nchunks - 1) & 1
        wait_data(bl)

        @pl.when(nchunks >= 3)
        def _():
            wait_out(bl)

        compute(bl)
        issue_out(nchunks - 1, bl)
        wait_out(bl)

        @pl.when(nchunks >= 2)
        def _():
            wait_out(1 - bl)

    return edge_kernel


def kernel(diag, tril, row, col):
    n, d = diag.shape
    e = tril.shape[0]
    dsi, diag_maps = _compute_dsi(diag)
    dsi8 = jnp.zeros((n, 8), jnp.float32).at[:, :d].set(dsi)

    assert e % _GRP == 0
    gb = e // _GRP
    trilf = tril.reshape(gb, _GRP, d).transpose(0, 2, 1).reshape(-1)

    edge_kernel = _sc_edge_kernel(e, d, 16)
    outf = edge_kernel(dsi8, trilf, row, col)
    out = outf.reshape(gb, d, _GRP).transpose(0, 2, 1).reshape(e, d)
    return diag_maps, out
